# Initial kernel scaffold; baseline (speedup 1.0000x reference)
import jax
import jax.numpy as jnp
from jax.experimental import pallas as pl


def kernel(x, edge_index, edge_attr, batch, W_n, b_n, W_e, lin_W, lin_b):
    raise NotImplementedError("write your pallas kernel here")


# speedup vs baseline: 2.3792x; 2.3792x over previous
import functools

import jax
import jax.numpy as jnp
from jax import lax
from jax.experimental import pallas as pl
from jax.experimental.pallas import tpu as pltpu
from jax.experimental.pallas import tpu_sc as plsc

N = 10000
E = 320000
D = 128
DH = D // 2
DEP = 16
L = 3
G = 128
C = 2

NC = 2
NS = 16
EPT = E // NS
CH = 80
NCHUNK = EPT // CH
RPT = 624
RPT_LAST = N - (NS - 1) * RPT

NB = 10
BR = N // NB


def _sc_agg_body(with_ea, *refs):
    if with_ea:
        (hla_hbm, hlb_hbm, src_hbm, dst_hbm, ea_hbm, zrows_hbm, zea_hbm,
         outa_hbm, outb_hbm, eaout_hbm,
         idx_s, idx_d, rows, ea_rows, bounce, ea_bounce, agg_sh, ea_sh,
         gsem) = refs
    else:
        (hla_hbm, hlb_hbm, src_hbm, dst_hbm, zrows_hbm,
         outa_hbm, outb_hbm,
         idx_s, idx_d, rows, bounce, agg_sh, gsem) = refs

    c = lax.axis_index("c")
    s = lax.axis_index("s")

    pltpu.sync_copy(zrows_hbm, bounce)
    if with_ea:
        pltpu.sync_copy(zea_hbm, ea_bounce)

    @pl.when(s < NS - 1)
    def _():
        pltpu.sync_copy(bounce.at[pl.ds(0, RPT)], agg_sh.at[pl.ds(s * RPT, RPT)])
        if with_ea:
            pltpu.sync_copy(
                ea_bounce.at[pl.ds(0, RPT)], ea_sh.at[pl.ds(s * RPT, RPT)]
            )

    @pl.when(s == NS - 1)
    def _():
        pltpu.sync_copy(bounce, agg_sh.at[pl.ds((NS - 1) * RPT, RPT_LAST)])
        if with_ea:
            pltpu.sync_copy(ea_bounce, ea_sh.at[pl.ds((NS - 1) * RPT, RPT_LAST)])

    plsc.subcore_barrier()

    ebase = s * EPT

    @pl.loop(0, NCHUNK)
    def _(j):
        base = ebase + j * CH
        pltpu.sync_copy(src_hbm.at[pl.ds(base, CH)], idx_s.at[0])
        pltpu.sync_copy(dst_hbm.at[pl.ds(base, CH)], idx_d.at[0])

        @pl.when(c == 0)
        def _():
            pltpu.async_copy(hla_hbm.at[idx_s.at[0]], rows, gsem).wait()

        @pl.when(c == 1)
        def _():
            pltpu.async_copy(hlb_hbm.at[idx_s.at[0]], rows, gsem).wait()

        pltpu.sync_copy(rows, agg_sh.at[idx_d.at[0]], add=True)
        if with_ea:
            @pl.when(c == 0)
            def _():
                pltpu.sync_copy(ea_hbm.at[pl.ds(base, CH)], ea_rows)
                pltpu.sync_copy(ea_rows, ea_sh.at[idx_d.at[0]], add=True)

    plsc.subcore_barrier()

    def _writeback(row0, nrows):
        pltpu.sync_copy(agg_sh.at[pl.ds(row0, nrows)], bounce.at[pl.ds(0, nrows)])

        @pl.when(c == 0)
        def _():
            pltpu.sync_copy(bounce.at[pl.ds(0, nrows)], outa_hbm.at[pl.ds(row0, nrows)])
            if with_ea:
                pltpu.sync_copy(
                    ea_sh.at[pl.ds(row0, nrows)], ea_bounce.at[pl.ds(0, nrows)]
                )
                pltpu.sync_copy(
                    ea_bounce.at[pl.ds(0, nrows)], eaout_hbm.at[pl.ds(row0, nrows)]
                )

        @pl.when(c == 1)
        def _():
            pltpu.sync_copy(bounce.at[pl.ds(0, nrows)], outb_hbm.at[pl.ds(row0, nrows)])

    @pl.when(s < NS - 1)
    def _():
        _writeback(s * RPT, RPT)

    @pl.when(s == NS - 1)
    def _():
        _writeback((NS - 1) * RPT, RPT_LAST)


def _make_sc_agg(with_ea):
    mesh = plsc.VectorSubcoreMesh(core_axis_name="c", subcore_axis_name="s")
    out_type = [
        jax.ShapeDtypeStruct((N, DH), jnp.float32),
        jax.ShapeDtypeStruct((N, DH), jnp.float32),
    ]
    scratch = [
        pltpu.VMEM((1, CH), jnp.int32),
        pltpu.VMEM((1, CH), jnp.int32),
        pltpu.VMEM((CH, DH), jnp.float32),
    ]
    if with_ea:
        out_type.append(jax.ShapeDtypeStruct((N, DEP), jnp.float32))
        scratch.append(pltpu.VMEM((CH, DEP), jnp.float32))
    scratch.append(pltpu.VMEM((RPT_LAST, DH), jnp.float32))
    if with_ea:
        scratch.append(pltpu.VMEM((RPT_LAST, DEP), jnp.float32))
    scratch.append(pltpu.VMEM_SHARED((N, DH), jnp.float32))
    if with_ea:
        scratch.append(pltpu.VMEM_SHARED((N, DEP), jnp.float32))
    scratch.append(pltpu.SemaphoreType.DMA)
    return pl.kernel(
        functools.partial(_sc_agg_body, with_ea),
        out_type=out_type,
        mesh=mesh,
        scratch_types=scratch,
        compiler_params=pltpu.CompilerParams(use_tc_tiling_on_sc=False),
    )


def _split_out(res, outa_b, outb_b):
    outa_b[...] = res[:, :DH]
    outb_b[...] = res[:, DH:]


def _tc_matmul0(x_b, w_b, b_b, outa_b, outb_b):
    res = jnp.dot(x_b[...], w_b[...], preferred_element_type=jnp.float32) + b_b[...]
    _split_out(res, outa_b, outb_b)


def _tc_combine(pa_b, pb_b, hla_b, hlb_b, ea_b, we_b, wn_b, bn_b, outa_b, outb_b):
    h = (
        jnp.concatenate([pa_b[...] + hla_b[...], pb_b[...] + hlb_b[...]], axis=1)
        + jnp.dot(ea_b[...], we_b[...], preferred_element_type=jnp.float32)
    )
    h = jnp.maximum(h, 0.0)
    res = jnp.dot(h, wn_b[...], preferred_element_type=jnp.float32) + bn_b[...]
    _split_out(res, outa_b, outb_b)


def _tc_final(pa_b, pb_b, hla_b, hlb_b, ea_b, we_b, batch_b, linw_b, linb_b,
              out_b, pooled):
    j = pl.program_id(0)
    h3 = (
        jnp.concatenate([pa_b[...] + hla_b[...], pb_b[...] + hlb_b[...]], axis=1)
        + jnp.dot(ea_b[...], we_b[...], preferred_element_type=jnp.float32)
    )
    b = batch_b[0, 0]
    m = (b[:, None] == lax.broadcasted_iota(jnp.int32, (1, G), 1)).astype(jnp.float32)
    contrib = lax.dot_general(
        m, h3, (((0,), (0,)), ((), ())), preferred_element_type=jnp.float32
    )

    @pl.when(j == 0)
    def _():
        pooled[...] = jnp.zeros_like(pooled)

    pooled[...] += contrib

    @pl.when(j == NB - 1)
    def _():
        out_b[...] = (
            jnp.dot(pooled[...], linw_b[...], preferred_element_type=jnp.float32)
            + linb_b[...]
        )


_row_spec = pl.BlockSpec((BR, D), lambda i: (i, 0))
_half_spec = pl.BlockSpec((BR, DH), lambda i: (i, 0))
_ea_spec = pl.BlockSpec((BR, DEP), lambda i: (i, 0))
_full = lambda shape: pl.BlockSpec(shape, lambda i: tuple(0 for _ in shape))

_half_out2 = [
    jax.ShapeDtypeStruct((N, DH), jnp.float32),
    jax.ShapeDtypeStruct((N, DH), jnp.float32),
]


def _matmul0(x, w, b):
    return pl.pallas_call(
        _tc_matmul0,
        grid=(NB,),
        in_specs=[_row_spec, _full((D, D)), _full((1, D))],
        out_specs=[_half_spec, _half_spec],
        out_shape=_half_out2,
    )(x, w, b)


def _combine_call(pa, pb, hla, hlb, ea, we, wn, bn):
    return pl.pallas_call(
        _tc_combine,
        grid=(NB,),
        in_specs=[
            _half_spec, _half_spec, _half_spec, _half_spec, _ea_spec,
            _full((DEP, D)), _full((D, D)), _full((1, D)),
        ],
        out_specs=[_half_spec, _half_spec],
        out_shape=_half_out2,
    )(pa, pb, hla, hlb, ea, we, wn, bn)


def _final_call(pa, pb, hla, hlb, ea, we, batch3, lin_W, lin_b2):
    return pl.pallas_call(
        _tc_final,
        grid=(NB,),
        in_specs=[
            _half_spec, _half_spec, _half_spec, _half_spec, _ea_spec,
            _full((DEP, D)),
            pl.BlockSpec((1, 1, BR), lambda i: (i, 0, 0)),
            _full((D, C)), _full((1, C)),
        ],
        out_specs=pl.BlockSpec((G, C), lambda i: (0, 0)),
        out_shape=jax.ShapeDtypeStruct((G, C), jnp.float32),
        scratch_shapes=[pltpu.VMEM((G, D), jnp.float32)],
    )(pa, pb, hla, hlb, ea, we, batch3, lin_W, lin_b2)


_sc_agg_ea = _make_sc_agg(True)
_sc_agg = _make_sc_agg(False)


def kernel(x, edge_index, edge_attr, batch, W_n, b_n, W_e, lin_W, lin_b):
    ei = edge_index.astype(jnp.int32)
    src = ei[0]
    dst = ei[1]
    ea_pad = jnp.concatenate(
        [edge_attr.astype(jnp.float32), jnp.zeros((E, DEP - 4), jnp.float32)], axis=1
    )
    we_pad = jnp.concatenate(
        [W_e.astype(jnp.float32), jnp.zeros((L, DEP - 4, D), jnp.float32)], axis=1
    )
    batch3 = batch.astype(jnp.int32).reshape(NB, 1, BR)
    zrows = jnp.zeros((RPT_LAST, DH), jnp.float32)
    zea = jnp.zeros((RPT_LAST, DEP), jnp.float32)
    bn2 = b_n.reshape(L, 1, D)
    linb2 = lin_b.reshape(1, C)

    hla, hlb = _matmul0(x, W_n[0], bn2[0])
    pa, pb, ea_agg = _sc_agg_ea(hla, hlb, src, dst, ea_pad, zrows, zea)
    hla, hlb = _combine_call(pa, pb, hla, hlb, ea_agg, we_pad[0], W_n[1], bn2[1])
    pa, pb = _sc_agg(hla, hlb, src, dst, zrows)
    hla, hlb = _combine_call(pa, pb, hla, hlb, ea_agg, we_pad[1], W_n[2], bn2[2])
    pa, pb = _sc_agg(hla, hlb, src, dst, zrows)
    out = _final_call(pa, pb, hla, hlb, ea_agg, we_pad[2], batch3, lin_W, linb2)
    return out


# speedup vs baseline: 4.5611x; 1.9171x over previous
import functools

import jax
import jax.numpy as jnp
from jax import lax
from jax.experimental import pallas as pl
from jax.experimental.pallas import tpu as pltpu
from jax.experimental.pallas import tpu_sc as plsc

N = 10000
E = 320000
D = 128
DH = D // 2
DEP = 16
L = 3
G = 128
C = 2

NC = 2
NS = 16
EPT = E // NS
CH = 80
NCHUNK = EPT // CH
RPT = 624
RPT_LAST = N - (NS - 1) * RPT
BNC = 320

NB = 10
BR = N // NB


def _sc_agg_body(hla_hbm, hlb_hbm, src_hbm, dst_hbm, zrows_hbm,
                 outa_hbm, outb_hbm,
                 idx_s, idx_d, rows0, rows1, bounce, agg_sh, gsem0, gsem1):
    rows_b = (rows0, rows1)
    gsem_b = (gsem0, gsem1)

    c = lax.axis_index("c")
    s = lax.axis_index("s")

    pltpu.sync_copy(zrows_hbm, bounce)

    def _zero(row0, nrows):
        for off in range(0, nrows, BNC):
            n = min(BNC, nrows - off)
            pltpu.sync_copy(
                bounce.at[pl.ds(0, n)], agg_sh.at[pl.ds(row0 + off, n)]
            )

    @pl.when(s < NS - 1)
    def _():
        _zero(s * RPT, RPT)

    @pl.when(s == NS - 1)
    def _():
        _zero((NS - 1) * RPT, RPT_LAST)

    pltpu.sync_copy(src_hbm.at[s], idx_s)
    pltpu.sync_copy(dst_hbm.at[s], idx_d)

    plsc.subcore_barrier()

    def _start_gather(j, b):
        @pl.when(c == 0)
        def _():
            pltpu.async_copy(hla_hbm.at[idx_s.at[j]], rows_b[b], gsem_b[b])

        @pl.when(c == 1)
        def _():
            pltpu.async_copy(hlb_hbm.at[idx_s.at[j]], rows_b[b], gsem_b[b])

    def _wait_gather(b):
        pltpu.make_async_copy(hla_hbm.at[pl.ds(0, CH)], rows_b[b], gsem_b[b]).wait()

    _start_gather(0, 0)

    @pl.loop(0, NCHUNK // 2)
    def _(g):
        for b in range(2):
            j = g * 2 + b
            _wait_gather(b)

            @pl.when(j + 1 < NCHUNK)
            def _():
                _start_gather(j + 1, 1 - b)

            pltpu.sync_copy(rows_b[b], agg_sh.at[idx_d.at[j]], add=True)

    plsc.subcore_barrier()

    def _writeback(row0, nrows):
        for off in range(0, nrows, BNC):
            n = min(BNC, nrows - off)
            pltpu.sync_copy(
                agg_sh.at[pl.ds(row0 + off, n)], bounce.at[pl.ds(0, n)]
            )

            @pl.when(c == 0)
            def _():
                pltpu.sync_copy(
                    bounce.at[pl.ds(0, n)], outa_hbm.at[pl.ds(row0 + off, n)]
                )

            @pl.when(c == 1)
            def _():
                pltpu.sync_copy(
                    bounce.at[pl.ds(0, n)], outb_hbm.at[pl.ds(row0 + off, n)]
                )

    @pl.when(s < NS - 1)
    def _():
        _writeback(s * RPT, RPT)

    @pl.when(s == NS - 1)
    def _():
        _writeback((NS - 1) * RPT, RPT_LAST)


def _sc_ea_body(dst_hbm, ea_hbm, zea_hbm, eaout_hbm,
                idx_d, ea0, ea1, ea_bounce, ea_sh, esem0, esem1):
    ea_b = (ea0, ea1)
    esem_b = (esem0, esem1)
    c = lax.axis_index("c")
    s = lax.axis_index("s")

    @pl.when(c == 0)
    def _():
        pltpu.sync_copy(zea_hbm, ea_bounce)

        @pl.when(s < NS - 1)
        def _():
            pltpu.sync_copy(
                ea_bounce.at[pl.ds(0, RPT)], ea_sh.at[pl.ds(s * RPT, RPT)]
            )

        @pl.when(s == NS - 1)
        def _():
            pltpu.sync_copy(ea_bounce, ea_sh.at[pl.ds((NS - 1) * RPT, RPT_LAST)])

        pltpu.sync_copy(dst_hbm.at[s], idx_d)

    plsc.subcore_barrier()

    @pl.when(c == 0)
    def _():
        def _start_load(j, b):
            pltpu.async_copy(
                ea_hbm.at[pl.ds(s * EPT + j * CH, CH)], ea_b[b], esem_b[b]
            )

        def _wait_load(b):
            pltpu.make_async_copy(ea_hbm.at[pl.ds(0, CH)], ea_b[b], esem_b[b]).wait()

        _start_load(0, 0)

        @pl.loop(0, NCHUNK // 2)
        def _(g):
            for b in range(2):
                j = g * 2 + b
                _wait_load(b)

                @pl.when(j + 1 < NCHUNK)
                def _():
                    _start_load(j + 1, 1 - b)

                pltpu.sync_copy(ea_b[b], ea_sh.at[idx_d.at[j]], add=True)

    plsc.subcore_barrier()

    @pl.when(c == 0)
    def _():
        @pl.when(s < NS - 1)
        def _():
            row0 = s * RPT
            pltpu.sync_copy(ea_sh.at[pl.ds(row0, RPT)], ea_bounce.at[pl.ds(0, RPT)])
            pltpu.sync_copy(ea_bounce.at[pl.ds(0, RPT)], eaout_hbm.at[pl.ds(row0, RPT)])

        @pl.when(s == NS - 1)
        def _():
            row0 = (NS - 1) * RPT
            pltpu.sync_copy(ea_sh.at[pl.ds(row0, RPT_LAST)], ea_bounce)
            pltpu.sync_copy(ea_bounce, eaout_hbm.at[pl.ds(row0, RPT_LAST)])


_sc_mesh_kw = dict(
    mesh=plsc.VectorSubcoreMesh(core_axis_name="c", subcore_axis_name="s"),
    compiler_params=pltpu.CompilerParams(use_tc_tiling_on_sc=False),
)

_sc_agg = pl.kernel(
    _sc_agg_body,
    out_type=[
        jax.ShapeDtypeStruct((N, DH), jnp.float32),
        jax.ShapeDtypeStruct((N, DH), jnp.float32),
    ],
    scratch_types=[
        pltpu.VMEM((NCHUNK, CH), jnp.int32),
        pltpu.VMEM((NCHUNK, CH), jnp.int32),
        pltpu.VMEM((CH, DH), jnp.float32),
        pltpu.VMEM((CH, DH), jnp.float32),
        pltpu.VMEM((BNC, DH), jnp.float32),
        pltpu.VMEM_SHARED((N, DH), jnp.float32),
        pltpu.SemaphoreType.DMA,
        pltpu.SemaphoreType.DMA,
    ],
    **_sc_mesh_kw,
)

_sc_ea = pl.kernel(
    _sc_ea_body,
    out_type=[jax.ShapeDtypeStruct((N, DEP), jnp.float32)],
    scratch_types=[
        pltpu.VMEM((NCHUNK, CH), jnp.int32),
        pltpu.VMEM((CH, DEP), jnp.float32),
        pltpu.VMEM((CH, DEP), jnp.float32),
        pltpu.VMEM((RPT_LAST, DEP), jnp.float32),
        pltpu.VMEM_SHARED((N, DEP), jnp.float32),
        pltpu.SemaphoreType.DMA,
        pltpu.SemaphoreType.DMA,
    ],
    **_sc_mesh_kw,
)


def _split_out(res, outa_b, outb_b):
    outa_b[...] = res[:, :DH]
    outb_b[...] = res[:, DH:]


def _tc_matmul0(x_b, w_b, b_b, outa_b, outb_b):
    res = jnp.dot(x_b[...], w_b[...], preferred_element_type=jnp.float32) + b_b[...]
    _split_out(res, outa_b, outb_b)


def _tc_combine(pa_b, pb_b, hla_b, hlb_b, ea_b, we_b, wn_b, bn_b, outa_b, outb_b):
    h = (
        jnp.concatenate([pa_b[...] + hla_b[...], pb_b[...] + hlb_b[...]], axis=1)
        + jnp.dot(ea_b[...], we_b[...], preferred_element_type=jnp.float32)
    )
    h = jnp.maximum(h, 0.0)
    res = jnp.dot(h, wn_b[...], preferred_element_type=jnp.float32) + bn_b[...]
    _split_out(res, outa_b, outb_b)


def _tc_final(pa_b, pb_b, hla_b, hlb_b, ea_b, we_b, batch_b, linw_b, linb_b,
              out_b, pooled):
    j = pl.program_id(0)
    h3 = (
        jnp.concatenate([pa_b[...] + hla_b[...], pb_b[...] + hlb_b[...]], axis=1)
        + jnp.dot(ea_b[...], we_b[...], preferred_element_type=jnp.float32)
    )
    b = batch_b[0, 0]
    m = (b[:, None] == lax.broadcasted_iota(jnp.int32, (1, G), 1)).astype(jnp.float32)
    contrib = lax.dot_general(
        m, h3, (((0,), (0,)), ((), ())), preferred_element_type=jnp.float32
    )

    @pl.when(j == 0)
    def _():
        pooled[...] = jnp.zeros_like(pooled)

    pooled[...] += contrib

    @pl.when(j == NB - 1)
    def _():
        out_b[...] = (
            jnp.dot(pooled[...], linw_b[...], preferred_element_type=jnp.float32)
            + linb_b[...]
        )


_row_spec = pl.BlockSpec((BR, D), lambda i: (i, 0))
_half_spec = pl.BlockSpec((BR, DH), lambda i: (i, 0))
_ea_spec = pl.BlockSpec((BR, DEP), lambda i: (i, 0))
_full = lambda shape: pl.BlockSpec(shape, lambda i: tuple(0 for _ in shape))

_half_out2 = [
    jax.ShapeDtypeStruct((N, DH), jnp.float32),
    jax.ShapeDtypeStruct((N, DH), jnp.float32),
]


def _matmul0(x, w, b):
    return pl.pallas_call(
        _tc_matmul0,
        grid=(NB,),
        in_specs=[_row_spec, _full((D, D)), _full((1, D))],
        out_specs=[_half_spec, _half_spec],
        out_shape=_half_out2,
    )(x, w, b)


def _combine_call(pa, pb, hla, hlb, ea, we, wn, bn):
    return pl.pallas_call(
        _tc_combine,
        grid=(NB,),
        in_specs=[
            _half_spec, _half_spec, _half_spec, _half_spec, _ea_spec,
            _full((DEP, D)), _full((D, D)), _full((1, D)),
        ],
        out_specs=[_half_spec, _half_spec],
        out_shape=_half_out2,
    )(pa, pb, hla, hlb, ea, we, wn, bn)


def _final_call(pa, pb, hla, hlb, ea, we, batch3, lin_W, lin_b2):
    return pl.pallas_call(
        _tc_final,
        grid=(NB,),
        in_specs=[
            _half_spec, _half_spec, _half_spec, _half_spec, _ea_spec,
            _full((DEP, D)),
            pl.BlockSpec((1, 1, BR), lambda i: (i, 0, 0)),
            _full((D, C)), _full((1, C)),
        ],
        out_specs=pl.BlockSpec((G, C), lambda i: (0, 0)),
        out_shape=jax.ShapeDtypeStruct((G, C), jnp.float32),
        scratch_shapes=[pltpu.VMEM((G, D), jnp.float32)],
    )(pa, pb, hla, hlb, ea, we, batch3, lin_W, lin_b2)


def kernel(x, edge_index, edge_attr, batch, W_n, b_n, W_e, lin_W, lin_b):
    ei = edge_index.astype(jnp.int32)
    src = ei[0].reshape(NS, NCHUNK, CH)
    dst = ei[1].reshape(NS, NCHUNK, CH)
    ea_pad = jnp.concatenate(
        [edge_attr.astype(jnp.float32), jnp.zeros((E, DEP - 4), jnp.float32)], axis=1
    )
    we_pad = jnp.concatenate(
        [W_e.astype(jnp.float32), jnp.zeros((L, DEP - 4, D), jnp.float32)], axis=1
    )
    batch3 = batch.astype(jnp.int32).reshape(NB, 1, BR)
    zrows = jnp.zeros((BNC, DH), jnp.float32)
    zea = jnp.zeros((RPT_LAST, DEP), jnp.float32)
    bn2 = b_n.reshape(L, 1, D)
    linb2 = lin_b.reshape(1, C)

    (ea_agg,) = _sc_ea(dst, ea_pad, zea)
    hla, hlb = _matmul0(x, W_n[0], bn2[0])
    pa, pb = _sc_agg(hla, hlb, src, dst, zrows)
    hla, hlb = _combine_call(pa, pb, hla, hlb, ea_agg, we_pad[0], W_n[1], bn2[1])
    pa, pb = _sc_agg(hla, hlb, src, dst, zrows)
    hla, hlb = _combine_call(pa, pb, hla, hlb, ea_agg, we_pad[1], W_n[2], bn2[2])
    pa, pb = _sc_agg(hla, hlb, src, dst, zrows)
    out = _final_call(pa, pb, hla, hlb, ea_agg, we_pad[2], batch3, lin_W, linb2)
    return out


# speedup vs baseline: 5.5921x; 1.2260x over previous
import functools

import jax
import jax.numpy as jnp
from jax import lax
from jax.experimental import pallas as pl
from jax.experimental.pallas import tpu as pltpu
from jax.experimental.pallas import tpu_sc as plsc

N = 10000
E = 320000
D = 128
DH = D // 2
DEP = 16
L = 3
G = 128
C = 2

NC = 2
NS = 16
EPT = E // NS
CH = 80
NCHUNK = EPT // CH
RPT = 624
RPT_LAST = N - (NS - 1) * RPT
BNC = 160
NBUF = 4

NB = 10
BR = N // NB


def _sc_agg_body(hla_hbm, hlb_hbm, src_hbm, dst_hbm, zrows_hbm,
                 outa_hbm, outb_hbm,
                 idx_s, idx_d, rows0, rows1, rows2, rows3, bounce, agg_sh,
                 gsem0, gsem1, gsem2, gsem3, ssem0, ssem1, ssem2, ssem3):
    rows_b = (rows0, rows1, rows2, rows3)
    gsem_b = (gsem0, gsem1, gsem2, gsem3)
    ssem_b = (ssem0, ssem1, ssem2, ssem3)

    c = lax.axis_index("c")
    s = lax.axis_index("s")

    pltpu.sync_copy(zrows_hbm, bounce)

    def _zero(row0, nrows):
        for off in range(0, nrows, BNC):
            n = min(BNC, nrows - off)
            pltpu.sync_copy(
                bounce.at[pl.ds(0, n)], agg_sh.at[pl.ds(row0 + off, n)]
            )

    @pl.when(s < NS - 1)
    def _():
        _zero(s * RPT, RPT)

    @pl.when(s == NS - 1)
    def _():
        _zero((NS - 1) * RPT, RPT_LAST)

    pltpu.sync_copy(src_hbm.at[s], idx_s)
    pltpu.sync_copy(dst_hbm.at[s], idx_d)

    plsc.subcore_barrier()

    def _start_gather(j, b):
        @pl.when(c == 0)
        def _():
            pltpu.async_copy(hla_hbm.at[idx_s.at[j]], rows_b[b], gsem_b[b])

        @pl.when(c == 1)
        def _():
            pltpu.async_copy(hlb_hbm.at[idx_s.at[j]], rows_b[b], gsem_b[b])

    def _wait_gather(b):
        pltpu.make_async_copy(hla_hbm.at[pl.ds(0, CH)], rows_b[b], gsem_b[b]).wait()

    def _start_scatter(j, b):
        pltpu.async_copy(rows_b[b], agg_sh.at[idx_d.at[j]], ssem_b[b], add=True)

    def _wait_scatter(b):
        pltpu.make_async_copy(rows_b[b], agg_sh.at[pl.ds(0, CH)], ssem_b[b]).wait()

    _start_gather(0, 0)
    _start_gather(1, 1)

    @pl.loop(0, NCHUNK // NBUF)
    def _(g):
        for b in range(NBUF):
            j = g * NBUF + b
            _wait_gather(b)
            _start_scatter(j, b)
            bn = (b + 2) % NBUF

            @pl.when(j >= 2)
            def _():
                _wait_scatter(bn)

            @pl.when(j + 2 < NCHUNK)
            def _():
                _start_gather(j + 2, bn)

    for j, b in ((NCHUNK - 2, (NCHUNK - 2) % NBUF), (NCHUNK - 1, (NCHUNK - 1) % NBUF)):
        _wait_gather(b)
        _start_scatter(j, b)
        _wait_scatter((b + 2) % NBUF)
    _wait_scatter((NCHUNK - 2) % NBUF)
    _wait_scatter((NCHUNK - 1) % NBUF)

    plsc.subcore_barrier()

    def _writeback(row0, nrows):
        for off in range(0, nrows, BNC):
            n = min(BNC, nrows - off)
            pltpu.sync_copy(
                agg_sh.at[pl.ds(row0 + off, n)], bounce.at[pl.ds(0, n)]
            )

            @pl.when(c == 0)
            def _():
                pltpu.sync_copy(
                    bounce.at[pl.ds(0, n)], outa_hbm.at[pl.ds(row0 + off, n)]
                )

            @pl.when(c == 1)
            def _():
                pltpu.sync_copy(
                    bounce.at[pl.ds(0, n)], outb_hbm.at[pl.ds(row0 + off, n)]
                )

    @pl.when(s < NS - 1)
    def _():
        _writeback(s * RPT, RPT)

    @pl.when(s == NS - 1)
    def _():
        _writeback((NS - 1) * RPT, RPT_LAST)


def _sc_ea_body(dst_hbm, ea_hbm, zea_hbm, eaout_hbm,
                idx_d, ea0, ea1, ea_bounce, ea_sh, esem0, esem1):
    ea_b = (ea0, ea1)
    esem_b = (esem0, esem1)
    c = lax.axis_index("c")
    s = lax.axis_index("s")

    @pl.when(c == 0)
    def _():
        pltpu.sync_copy(zea_hbm, ea_bounce)

        @pl.when(s < NS - 1)
        def _():
            pltpu.sync_copy(
                ea_bounce.at[pl.ds(0, RPT)], ea_sh.at[pl.ds(s * RPT, RPT)]
            )

        @pl.when(s == NS - 1)
        def _():
            pltpu.sync_copy(ea_bounce, ea_sh.at[pl.ds((NS - 1) * RPT, RPT_LAST)])

        pltpu.sync_copy(dst_hbm.at[s], idx_d)

    plsc.subcore_barrier()

    @pl.when(c == 0)
    def _():
        def _start_load(j, b):
            pltpu.async_copy(
                ea_hbm.at[pl.ds(s * EPT + j * CH, CH)], ea_b[b], esem_b[b]
            )

        def _wait_load(b):
            pltpu.make_async_copy(ea_hbm.at[pl.ds(0, CH)], ea_b[b], esem_b[b]).wait()

        _start_load(0, 0)

        @pl.loop(0, NCHUNK // 2)
        def _(g):
            for b in range(2):
                j = g * 2 + b
                _wait_load(b)

                @pl.when(j + 1 < NCHUNK)
                def _():
                    _start_load(j + 1, 1 - b)

                pltpu.sync_copy(ea_b[b], ea_sh.at[idx_d.at[j]], add=True)

    plsc.subcore_barrier()

    @pl.when(c == 0)
    def _():
        @pl.when(s < NS - 1)
        def _():
            row0 = s * RPT
            pltpu.sync_copy(ea_sh.at[pl.ds(row0, RPT)], ea_bounce.at[pl.ds(0, RPT)])
            pltpu.sync_copy(ea_bounce.at[pl.ds(0, RPT)], eaout_hbm.at[pl.ds(row0, RPT)])

        @pl.when(s == NS - 1)
        def _():
            row0 = (NS - 1) * RPT
            pltpu.sync_copy(ea_sh.at[pl.ds(row0, RPT_LAST)], ea_bounce)
            pltpu.sync_copy(ea_bounce, eaout_hbm.at[pl.ds(row0, RPT_LAST)])


_sc_mesh_kw = dict(
    mesh=plsc.VectorSubcoreMesh(core_axis_name="c", subcore_axis_name="s"),
    compiler_params=pltpu.CompilerParams(use_tc_tiling_on_sc=False),
)

_sc_agg = pl.kernel(
    _sc_agg_body,
    out_type=[
        jax.ShapeDtypeStruct((N, DH), jnp.float32),
        jax.ShapeDtypeStruct((N, DH), jnp.float32),
    ],
    scratch_types=(
        [
            pltpu.VMEM((NCHUNK, CH), jnp.int32),
            pltpu.VMEM((NCHUNK, CH), jnp.int32),
        ]
        + [pltpu.VMEM((CH, DH), jnp.float32)] * NBUF
        + [pltpu.VMEM((BNC, DH), jnp.float32)]
        + [pltpu.VMEM_SHARED((N, DH), jnp.float32)]
        + [pltpu.SemaphoreType.DMA] * (2 * NBUF)
    ),
    **_sc_mesh_kw,
)

_sc_ea = pl.kernel(
    _sc_ea_body,
    out_type=[jax.ShapeDtypeStruct((N, DEP), jnp.float32)],
    scratch_types=[
        pltpu.VMEM((NCHUNK, CH), jnp.int32),
        pltpu.VMEM((CH, DEP), jnp.float32),
        pltpu.VMEM((CH, DEP), jnp.float32),
        pltpu.VMEM((RPT_LAST, DEP), jnp.float32),
        pltpu.VMEM_SHARED((N, DEP), jnp.float32),
        pltpu.SemaphoreType.DMA,
        pltpu.SemaphoreType.DMA,
    ],
    **_sc_mesh_kw,
)


def _split_out(res, outa_b, outb_b):
    outa_b[...] = res[:, :DH]
    outb_b[...] = res[:, DH:]


def _tc_matmul0(x_b, w_b, b_b, outa_b, outb_b):
    res = jnp.dot(x_b[...], w_b[...], preferred_element_type=jnp.float32) + b_b[...]
    _split_out(res, outa_b, outb_b)


def _tc_combine(pa_b, pb_b, hla_b, hlb_b, ea_b, we_b, wn_b, bn_b, outa_b, outb_b):
    h = (
        jnp.concatenate([pa_b[...] + hla_b[...], pb_b[...] + hlb_b[...]], axis=1)
        + jnp.dot(ea_b[...], we_b[...], preferred_element_type=jnp.float32)
    )
    h = jnp.maximum(h, 0.0)
    res = jnp.dot(h, wn_b[...], preferred_element_type=jnp.float32) + bn_b[...]
    _split_out(res, outa_b, outb_b)


def _tc_final(pa_b, pb_b, hla_b, hlb_b, ea_b, we_b, batch_b, linw_b, linb_b,
              out_b, pooled):
    j = pl.program_id(0)
    h3 = (
        jnp.concatenate([pa_b[...] + hla_b[...], pb_b[...] + hlb_b[...]], axis=1)
        + jnp.dot(ea_b[...], we_b[...], preferred_element_type=jnp.float32)
    )
    b = batch_b[0, 0]
    m = (b[:, None] == lax.broadcasted_iota(jnp.int32, (1, G), 1)).astype(jnp.float32)
    contrib = lax.dot_general(
        m, h3, (((0,), (0,)), ((), ())), preferred_element_type=jnp.float32
    )

    @pl.when(j == 0)
    def _():
        pooled[...] = jnp.zeros_like(pooled)

    pooled[...] += contrib

    @pl.when(j == NB - 1)
    def _():
        out_b[...] = (
            jnp.dot(pooled[...], linw_b[...], preferred_element_type=jnp.float32)
            + linb_b[...]
        )


_row_spec = pl.BlockSpec((BR, D), lambda i: (i, 0))
_half_spec = pl.BlockSpec((BR, DH), lambda i: (i, 0))
_ea_spec = pl.BlockSpec((BR, DEP), lambda i: (i, 0))
_full = lambda shape: pl.BlockSpec(shape, lambda i: tuple(0 for _ in shape))

_half_out2 = [
    jax.ShapeDtypeStruct((N, DH), jnp.float32),
    jax.ShapeDtypeStruct((N, DH), jnp.float32),
]


def _matmul0(x, w, b):
    return pl.pallas_call(
        _tc_matmul0,
        grid=(NB,),
        in_specs=[_row_spec, _full((D, D)), _full((1, D))],
        out_specs=[_half_spec, _half_spec],
        out_shape=_half_out2,
    )(x, w, b)


def _combine_call(pa, pb, hla, hlb, ea, we, wn, bn):
    return pl.pallas_call(
        _tc_combine,
        grid=(NB,),
        in_specs=[
            _half_spec, _half_spec, _half_spec, _half_spec, _ea_spec,
            _full((DEP, D)), _full((D, D)), _full((1, D)),
        ],
        out_specs=[_half_spec, _half_spec],
        out_shape=_half_out2,
    )(pa, pb, hla, hlb, ea, we, wn, bn)


def _final_call(pa, pb, hla, hlb, ea, we, batch3, lin_W, lin_b2):
    return pl.pallas_call(
        _tc_final,
        grid=(NB,),
        in_specs=[
            _half_spec, _half_spec, _half_spec, _half_spec, _ea_spec,
            _full((DEP, D)),
            pl.BlockSpec((1, 1, BR), lambda i: (i, 0, 0)),
            _full((D, C)), _full((1, C)),
        ],
        out_specs=pl.BlockSpec((G, C), lambda i: (0, 0)),
        out_shape=jax.ShapeDtypeStruct((G, C), jnp.float32),
        scratch_shapes=[pltpu.VMEM((G, D), jnp.float32)],
    )(pa, pb, hla, hlb, ea, we, batch3, lin_W, lin_b2)


def kernel(x, edge_index, edge_attr, batch, W_n, b_n, W_e, lin_W, lin_b):
    ei = edge_index.astype(jnp.int32)
    src = ei[0].reshape(NS, NCHUNK, CH)
    dst = ei[1].reshape(NS, NCHUNK, CH)
    ea_pad = jnp.concatenate(
        [edge_attr.astype(jnp.float32), jnp.zeros((E, DEP - 4), jnp.float32)], axis=1
    )
    we_pad = jnp.concatenate(
        [W_e.astype(jnp.float32), jnp.zeros((L, DEP - 4, D), jnp.float32)], axis=1
    )
    batch3 = batch.astype(jnp.int32).reshape(NB, 1, BR)
    zrows = jnp.zeros((BNC, DH), jnp.float32)
    zea = jnp.zeros((RPT_LAST, DEP), jnp.float32)
    bn2 = b_n.reshape(L, 1, D)
    linb2 = lin_b.reshape(1, C)

    (ea_agg,) = _sc_ea(dst, ea_pad, zea)
    hla, hlb = _matmul0(x, W_n[0], bn2[0])
    pa, pb = _sc_agg(hla, hlb, src, dst, zrows)
    hla, hlb = _combine_call(pa, pb, hla, hlb, ea_agg, we_pad[0], W_n[1], bn2[1])
    pa, pb = _sc_agg(hla, hlb, src, dst, zrows)
    hla, hlb = _combine_call(pa, pb, hla, hlb, ea_agg, we_pad[1], W_n[2], bn2[2])
    pa, pb = _sc_agg(hla, hlb, src, dst, zrows)
    out = _final_call(pa, pb, hla, hlb, ea_agg, we_pad[2], batch3, lin_W, linb2)
    return out
